# baseline (speedup 1.0000x reference)
import functools

import jax
import jax.numpy as jnp
import numpy as np
from jax import lax
from jax.experimental import pallas as pl
from jax.experimental.pallas import tpu as pltpu
from jax.experimental.pallas import tpu_sc as plsc

N = 48
D3 = 128
ED = 384
NW = 32
CH = 16
_BIG = np.float32(3.0e38)


def _sqrt16(x):
    xc = jnp.maximum(x, np.float32(1e-30))
    i = plsc.bitcast(xc, jnp.int32)
    i = np.int32(0x5F3759DF) - lax.shift_right_logical(i, 1)
    y = plsc.bitcast(i, jnp.float32)
    half = np.float32(0.5) * xc
    for _ in range(3):
        y = y * (np.float32(1.5) - half * y * y)
    return x * y


def _bcast_lane(v, idxv):
    dn = lax.GatherDimensionNumbers(
        offset_dims=(), collapsed_slice_dims=(0,), start_index_map=(0,))
    return lax.gather(v, idxv[:, None], dn, (1,),
                      mode=lax.GatherScatterMode.PROMISE_IN_BOUNDS)


@functools.lru_cache(maxsize=None)
def _make_sc_scalars(B):
    CPW = B // NW
    NCHUNK = CPW // CH
    mesh = plsc.VectorSubcoreMesh(core_axis_name="c", subcore_axis_name="s")

    @functools.partial(
        pl.kernel,
        out_type=(jax.ShapeDtypeStruct((B, N), jnp.float32),
                  jax.ShapeDtypeStruct((B, N), jnp.float32)),
        mesh=mesh,
        scratch_types=[
            pltpu.VMEM((CH * N * 3,), jnp.float32),
            pltpu.VMEM((CH, N), jnp.float32),
            pltpu.VMEM((CH, N), jnp.float32),
        ],
        compiler_params=pltpu.CompilerParams(needs_layout_passes=False),
    )
    def sc_scalars(pts_hbm, cd_hbm, ld_hbm, pts_v, cd_v, ld_v):
        wid = lax.axis_index("s") * 2 + lax.axis_index("c")
        base = wid * CPW
        iota = lax.iota(jnp.int32, 16)
        zero16 = iota * 0

        perm_idx = []
        seg_sel = []
        for q in range(3):
            f = iota * 3 + q
            perm_idx.append(f & 15)
            s = lax.shift_right_logical(f, 4)
            seg_sel.append((s == 0, s == 1))

        def cloud_body(ci, _):
            cbase = ci * (N * 3)
            v = [pts_v[pl.ds(cbase + 16 * k, 16)] for k in range(9)]
            xs, ys, zs = [], [], []
            for c in range(3):
                for q, out in ((0, xs), (1, ys), (2, zs)):
                    p0 = _bcast_lane(v[3 * c + 0], perm_idx[q])
                    p1 = _bcast_lane(v[3 * c + 1], perm_idx[q])
                    p2 = _bcast_lane(v[3 * c + 2], perm_idx[q])
                    is0, is1 = seg_sel[q]
                    out.append(jnp.where(is0, p0, jnp.where(is1, p1, p2)))
            inv_n = np.float32(1.0 / N)
            cx = jnp.sum(xs[0] + xs[1] + xs[2]) * inv_n
            cy = jnp.sum(ys[0] + ys[1] + ys[2]) * inv_n
            cz = jnp.sum(zs[0] + zs[1] + zs[2]) * inv_n
            big = zero16.astype(jnp.float32) + _BIG
            m1 = [big, big, big]
            m2 = [big, big, big]
            m3 = [big, big, big]
            for jc in range(3):
                for jl in range(16):
                    jidx = zero16 + jl
                    xj = _bcast_lane(xs[jc], jidx)
                    yj = _bcast_lane(ys[jc], jidx)
                    zj = _bcast_lane(zs[jc], jidx)
                    for ic in range(3):
                        dx = xs[ic] - xj
                        dy = ys[ic] - yj
                        dz = zs[ic] - zj
                        d = dx * dx + dy * dy + dz * dz
                        if ic == jc:
                            d = jnp.where(iota == jl, _BIG, d)
                        lo = jnp.minimum(m1[ic], d)
                        hi = jnp.maximum(m1[ic], d)
                        m1[ic] = lo
                        lo2 = jnp.minimum(m2[ic], hi)
                        hi2 = jnp.maximum(m2[ic], hi)
                        m2[ic] = lo2
                        m3[ic] = jnp.minimum(m3[ic], hi2)
            third = np.float32(1.0 / 3.0)
            for ic in range(3):
                dxc = xs[ic] - cx
                dyc = ys[ic] - cy
                dzc = zs[ic] - cz
                cdv = _sqrt16(dxc * dxc + dyc * dyc + dzc * dzc)
                ldv = (_sqrt16(m1[ic]) + _sqrt16(m2[ic]) + _sqrt16(m3[ic])) * third
                cd_v[ci, pl.ds(16 * ic, 16)] = cdv
                ld_v[ci, pl.ds(16 * ic, 16)] = ldv
            return 0

        def chunk_body(ck, _):
            b0 = base + ck * CH
            pltpu.sync_copy(pts_hbm.at[pl.ds(b0 * (N * 3), CH * N * 3)], pts_v)
            lax.fori_loop(0, CH, cloud_body, 0, unroll=False)
            pltpu.sync_copy(cd_v, cd_hbm.at[pl.ds(b0, CH)])
            pltpu.sync_copy(ld_v, ld_hbm.at[pl.ds(b0, CH)])
            return 0

        lax.fori_loop(0, NCHUNK, chunk_body, 0, unroll=False)

    return sc_scalars


@functools.lru_cache(maxsize=None)
def _make_tc_expand(B, BB=32):
    grid = (B // BB,)

    def body(cd_ref, ld_ref, wdist_ref, bdist_ref, emb_ref, wdens_ref,
             bdens_ref, wout_ref, bout_ref, out_ref):
        W = wout_ref[...]
        W1 = W[0:D3, :]
        W2 = W[D3:2 * D3, :]
        W3 = W[2 * D3:, :]
        f32 = jnp.float32
        hi = lax.Precision.HIGHEST
        v1 = jnp.dot(wdist_ref[...], W1, preferred_element_type=f32,
                     precision=hi)
        v2 = jnp.dot(wdens_ref[...], W3, preferred_element_type=f32,
                     precision=hi)
        cvec = (jnp.dot(bdist_ref[...].reshape(1, D3), W1,
                        preferred_element_type=f32, precision=hi)
                + jnp.dot(emb_ref[N:N + 1, :], W2,
                          preferred_element_type=f32, precision=hi)
                + jnp.dot(bdens_ref[...].reshape(1, D3), W3,
                          preferred_element_type=f32, precision=hi)
                + bout_ref[...].reshape(1, ED))
        cd3 = cd_ref[...][:, :, None]
        ld3 = ld_ref[...][:, :, None]
        out_ref[...] = (cd3 * v1.reshape(1, 1, ED)
                        + ld3 * v2.reshape(1, 1, ED)
                        + cvec.reshape(1, 1, ED))

    return pl.pallas_call(
        body,
        grid=grid,
        in_specs=[
            pl.BlockSpec((BB, N), lambda i: (i, 0)),
            pl.BlockSpec((BB, N), lambda i: (i, 0)),
            pl.BlockSpec((1, D3), lambda i: (0, 0)),
            pl.BlockSpec((D3,), lambda i: (0,)),
            pl.BlockSpec((50, D3), lambda i: (0, 0)),
            pl.BlockSpec((1, D3), lambda i: (0, 0)),
            pl.BlockSpec((D3,), lambda i: (0,)),
            pl.BlockSpec((ED, ED), lambda i: (0, 0)),
            pl.BlockSpec((ED,), lambda i: (0,)),
        ],
        out_specs=pl.BlockSpec((BB, N, ED), lambda i: (i, 0, 0)),
        out_shape=jax.ShapeDtypeStruct((B, N, ED), jnp.float32),
        compiler_params=pltpu.CompilerParams(
            dimension_semantics=("arbitrary",)),
    )


def kernel(points, W_dist, b_dist, emb_count, W_dens, b_dens, W_out, b_out):
    B = points.shape[0]
    cd, ld = _make_sc_scalars(B)(points.reshape(B * N * 3))
    return _make_tc_expand(B)(cd, ld, W_dist, b_dist, emb_count, W_dens,
                              b_dens, W_out, b_out)


# speedup vs baseline: 1.0502x; 1.0502x over previous
import functools

import jax
import jax.numpy as jnp
import numpy as np
from jax import lax
from jax.experimental import pallas as pl
from jax.experimental.pallas import tpu as pltpu
from jax.experimental.pallas import tpu_sc as plsc

N = 48
D3 = 128
ED = 384
NW = 32
CH = 16
_BIG = np.float32(3.0e38)


def _sqrt16(x):
    xc = jnp.maximum(x, np.float32(1e-30))
    i = plsc.bitcast(xc, jnp.int32)
    i = np.int32(0x5F3759DF) - lax.shift_right_logical(i, 1)
    y = plsc.bitcast(i, jnp.float32)
    half = np.float32(0.5) * xc
    for _ in range(3):
        y = y * (np.float32(1.5) - half * y * y)
    return x * y


def _bcast_lane(v, idxv):
    dn = lax.GatherDimensionNumbers(
        offset_dims=(), collapsed_slice_dims=(0,), start_index_map=(0,))
    return lax.gather(v, idxv[:, None], dn, (1,),
                      mode=lax.GatherScatterMode.PROMISE_IN_BOUNDS)


@functools.lru_cache(maxsize=None)
def _make_sc_scalars(B):
    CPW = B // NW
    NCHUNK = CPW // CH
    mesh = plsc.VectorSubcoreMesh(core_axis_name="c", subcore_axis_name="s")

    @functools.partial(
        pl.kernel,
        out_type=(jax.ShapeDtypeStruct((B, N), jnp.float32),
                  jax.ShapeDtypeStruct((B, N), jnp.float32)),
        mesh=mesh,
        scratch_types=[
            pltpu.VMEM((CH * N * 3,), jnp.float32),
            pltpu.VMEM((CH, N), jnp.float32),
            pltpu.VMEM((CH, N), jnp.float32),
        ],
        compiler_params=pltpu.CompilerParams(needs_layout_passes=False),
    )
    def sc_scalars(pts_hbm, cd_hbm, ld_hbm, pts_v, cd_v, ld_v):
        wid = lax.axis_index("s") * 2 + lax.axis_index("c")
        base = wid * CPW
        iota = lax.iota(jnp.int32, 16)
        zero16 = iota * 0

        perm_idx = []
        seg_sel = []
        for q in range(3):
            f = iota * 3 + q
            perm_idx.append(f & 15)
            s = lax.shift_right_logical(f, 4)
            seg_sel.append((s == 0, s == 1))

        def cloud_body(ci, _):
            cbase = ci * (N * 3)
            v = [pts_v[pl.ds(cbase + 16 * k, 16)] for k in range(9)]
            xs, ys, zs = [], [], []
            for c in range(3):
                for q, out in ((0, xs), (1, ys), (2, zs)):
                    p0 = _bcast_lane(v[3 * c + 0], perm_idx[q])
                    p1 = _bcast_lane(v[3 * c + 1], perm_idx[q])
                    p2 = _bcast_lane(v[3 * c + 2], perm_idx[q])
                    is0, is1 = seg_sel[q]
                    out.append(jnp.where(is0, p0, jnp.where(is1, p1, p2)))
            inv_n = np.float32(1.0 / N)
            cx = jnp.sum(xs[0] + xs[1] + xs[2]) * inv_n
            cy = jnp.sum(ys[0] + ys[1] + ys[2]) * inv_n
            cz = jnp.sum(zs[0] + zs[1] + zs[2]) * inv_n
            big = zero16.astype(jnp.float32) + _BIG
            m1 = [big, big, big]
            m2 = [big, big, big]
            m3 = [big, big, big]
            for jc in range(3):
                for jl in range(16):
                    jidx = zero16 + jl
                    xj = _bcast_lane(xs[jc], jidx)
                    yj = _bcast_lane(ys[jc], jidx)
                    zj = _bcast_lane(zs[jc], jidx)
                    for ic in range(3):
                        dx = xs[ic] - xj
                        dy = ys[ic] - yj
                        dz = zs[ic] - zj
                        d = dx * dx + dy * dy + dz * dz
                        if ic == jc:
                            d = jnp.where(iota == jl, _BIG, d)
                        lo = jnp.minimum(m1[ic], d)
                        hi = jnp.maximum(m1[ic], d)
                        m1[ic] = lo
                        lo2 = jnp.minimum(m2[ic], hi)
                        hi2 = jnp.maximum(m2[ic], hi)
                        m2[ic] = lo2
                        m3[ic] = jnp.minimum(m3[ic], hi2)
            third = np.float32(1.0 / 3.0)
            for ic in range(3):
                dxc = xs[ic] - cx
                dyc = ys[ic] - cy
                dzc = zs[ic] - cz
                cdv = _sqrt16(dxc * dxc + dyc * dyc + dzc * dzc)
                ldv = (_sqrt16(m1[ic]) + _sqrt16(m2[ic]) + _sqrt16(m3[ic])) * third
                cd_v[ci, pl.ds(16 * ic, 16)] = cdv
                ld_v[ci, pl.ds(16 * ic, 16)] = ldv
            return 0

        def chunk_body(ck, _):
            b0 = base + ck * CH
            pltpu.sync_copy(pts_hbm.at[pl.ds(b0 * (N * 3), CH * N * 3)], pts_v)
            lax.fori_loop(0, CH, cloud_body, 0, unroll=False)
            pltpu.sync_copy(cd_v, cd_hbm.at[pl.ds(b0, CH)])
            pltpu.sync_copy(ld_v, ld_hbm.at[pl.ds(b0, CH)])
            return 0

        lax.fori_loop(0, NCHUNK, chunk_body, 0, unroll=False)

    return sc_scalars


@functools.lru_cache(maxsize=None)
def _make_tc_expand(B, BB=32):
    grid = (B // BB,)

    def body(cd_ref, ld_ref, wdist_ref, bdist_ref, emb_ref, wdens_ref,
             bdens_ref, wout_ref, bout_ref, out_ref, fold_ref):
        @pl.when(pl.program_id(0) == 0)
        def _fold():
            W = wout_ref[...]
            W1 = W[0:D3, :]
            W2 = W[D3:2 * D3, :]
            W3 = W[2 * D3:, :]
            f32 = jnp.float32
            hi = lax.Precision.HIGHEST
            v1 = jnp.dot(wdist_ref[...], W1, preferred_element_type=f32,
                         precision=hi)
            v2 = jnp.dot(wdens_ref[...], W3, preferred_element_type=f32,
                         precision=hi)
            cvec = (jnp.dot(bdist_ref[...].reshape(1, D3), W1,
                            preferred_element_type=f32, precision=hi)
                    + jnp.dot(emb_ref[N:N + 1, :], W2,
                              preferred_element_type=f32, precision=hi)
                    + jnp.dot(bdens_ref[...].reshape(1, D3), W3,
                              preferred_element_type=f32, precision=hi)
                    + bout_ref[...].reshape(1, ED))
            fold_ref[0:1, :] = v1
            fold_ref[1:2, :] = v2
            fold_ref[2:3, :] = cvec

        cd3 = cd_ref[...][:, :, None]
        ld3 = ld_ref[...][:, :, None]
        out_ref[...] = (cd3 * fold_ref[0:1, :].reshape(1, 1, ED)
                        + ld3 * fold_ref[1:2, :].reshape(1, 1, ED)
                        + fold_ref[2:3, :].reshape(1, 1, ED))

    return pl.pallas_call(
        body,
        grid=grid,
        in_specs=[
            pl.BlockSpec((BB, N), lambda i: (i, 0)),
            pl.BlockSpec((BB, N), lambda i: (i, 0)),
            pl.BlockSpec((1, D3), lambda i: (0, 0)),
            pl.BlockSpec((D3,), lambda i: (0,)),
            pl.BlockSpec((50, D3), lambda i: (0, 0)),
            pl.BlockSpec((1, D3), lambda i: (0, 0)),
            pl.BlockSpec((D3,), lambda i: (0,)),
            pl.BlockSpec((ED, ED), lambda i: (0, 0)),
            pl.BlockSpec((ED,), lambda i: (0,)),
        ],
        out_specs=pl.BlockSpec((BB, N, ED), lambda i: (i, 0, 0)),
        out_shape=jax.ShapeDtypeStruct((B, N, ED), jnp.float32),
        scratch_shapes=[pltpu.VMEM((8, ED), jnp.float32)],
        compiler_params=pltpu.CompilerParams(
            dimension_semantics=("arbitrary",)),
    )


def kernel(points, W_dist, b_dist, emb_count, W_dens, b_dens, W_out, b_out):
    B = points.shape[0]
    cd, ld = _make_sc_scalars(B)(points.reshape(B * N * 3))
    return _make_tc_expand(B)(cd, ld, W_dist, b_dist, emb_count, W_dens,
                              b_dens, W_out, b_out)


# speedup vs baseline: 1.5374x; 1.4639x over previous
import functools

import jax
import jax.numpy as jnp
import numpy as np
from jax import lax
from jax.experimental import pallas as pl
from jax.experimental.pallas import tpu as pltpu
from jax.experimental.pallas import tpu_sc as plsc

N = 48
D3 = 128
ED = 384
NW = 32
CH = 16
_BIG = np.float32(3.0e38)


def _sqrt16(x):
    xc = jnp.maximum(x, np.float32(1e-30))
    i = plsc.bitcast(xc, jnp.int32)
    i = np.int32(0x5F3759DF) - lax.shift_right_logical(i, 1)
    y = plsc.bitcast(i, jnp.float32)
    half = np.float32(0.5) * xc
    for _ in range(3):
        y = y * (np.float32(1.5) - half * y * y)
    return x * y


def _bcast_lane(v, idxv):
    dn = lax.GatherDimensionNumbers(
        offset_dims=(), collapsed_slice_dims=(0,), start_index_map=(0,))
    return lax.gather(v, idxv[:, None], dn, (1,),
                      mode=lax.GatherScatterMode.PROMISE_IN_BOUNDS)


@functools.lru_cache(maxsize=None)
def _make_sc_scalars(B):
    CPW = B // NW
    NCHUNK = CPW // CH
    mesh = plsc.VectorSubcoreMesh(core_axis_name="c", subcore_axis_name="s")

    @functools.partial(
        pl.kernel,
        out_type=(jax.ShapeDtypeStruct((B, N), jnp.float32),
                  jax.ShapeDtypeStruct((B, N), jnp.float32)),
        mesh=mesh,
        scratch_types=[
            pltpu.VMEM((CH * N * 3,), jnp.float32),
            pltpu.VMEM((CH, N), jnp.float32),
            pltpu.VMEM((CH, N), jnp.float32),
        ],
        compiler_params=pltpu.CompilerParams(needs_layout_passes=False),
    )
    def sc_scalars(pts_hbm, cd_hbm, ld_hbm, pts_v, cd_v, ld_v):
        wid = lax.axis_index("s") * 2 + lax.axis_index("c")
        base = wid * CPW
        iota = lax.iota(jnp.int32, 16)
        zero16 = iota * 0

        def cloud_body(ci, _):
            cbase = ci * (N * 3)
            xs, ys, zs = [], [], []
            for c in range(3):
                off = cbase + 16 * c
                xs.append(pts_v[pl.ds(off, 16)])
                ys.append(pts_v[pl.ds(off + N, 16)])
                zs.append(pts_v[pl.ds(off + 2 * N, 16)])
            inv_n = np.float32(1.0 / N)
            cx = jnp.sum(xs[0] + xs[1] + xs[2]) * inv_n
            cy = jnp.sum(ys[0] + ys[1] + ys[2]) * inv_n
            cz = jnp.sum(zs[0] + zs[1] + zs[2]) * inv_n
            big = zero16.astype(jnp.float32) + _BIG
            m1 = [big, big, big]
            m2 = [big, big, big]
            m3 = [big, big, big]
            for jc in range(3):
                for jl in range(16):
                    jidx = zero16 + jl
                    xj = _bcast_lane(xs[jc], jidx)
                    yj = _bcast_lane(ys[jc], jidx)
                    zj = _bcast_lane(zs[jc], jidx)
                    for ic in range(3):
                        dx = xs[ic] - xj
                        dy = ys[ic] - yj
                        dz = zs[ic] - zj
                        d = dx * dx + dy * dy + dz * dz
                        if ic == jc:
                            d = jnp.where(iota == jl, _BIG, d)
                        lo = jnp.minimum(m1[ic], d)
                        hi = jnp.maximum(m1[ic], d)
                        m1[ic] = lo
                        lo2 = jnp.minimum(m2[ic], hi)
                        hi2 = jnp.maximum(m2[ic], hi)
                        m2[ic] = lo2
                        m3[ic] = jnp.minimum(m3[ic], hi2)
            third = np.float32(1.0 / 3.0)
            for ic in range(3):
                dxc = xs[ic] - cx
                dyc = ys[ic] - cy
                dzc = zs[ic] - cz
                cdv = _sqrt16(dxc * dxc + dyc * dyc + dzc * dzc)
                ldv = (_sqrt16(m1[ic]) + _sqrt16(m2[ic]) + _sqrt16(m3[ic])) * third
                cd_v[ci, pl.ds(16 * ic, 16)] = cdv
                ld_v[ci, pl.ds(16 * ic, 16)] = ldv
            return 0

        def chunk_body(ck, _):
            b0 = base + ck * CH
            pltpu.sync_copy(pts_hbm.at[pl.ds(b0 * (N * 3), CH * N * 3)], pts_v)
            lax.fori_loop(0, CH, cloud_body, 0, unroll=False)
            pltpu.sync_copy(cd_v, cd_hbm.at[pl.ds(b0, CH)])
            pltpu.sync_copy(ld_v, ld_hbm.at[pl.ds(b0, CH)])
            return 0

        lax.fori_loop(0, NCHUNK, chunk_body, 0, unroll=False)

    return sc_scalars


@functools.lru_cache(maxsize=None)
def _make_tc_expand(B, BB=32):
    grid = (B // BB,)

    def body(cd_ref, ld_ref, wdist_ref, bdist_ref, emb_ref, wdens_ref,
             bdens_ref, wout_ref, bout_ref, out_ref):
        W = wout_ref[...]
        W1 = W[0:D3, :]
        W2 = W[D3:2 * D3, :]
        W3 = W[2 * D3:, :]
        f32 = jnp.float32
        v1 = jnp.dot(wdist_ref[...], W1, preferred_element_type=f32)
        v2 = jnp.dot(wdens_ref[...], W3, preferred_element_type=f32)
        cvec = (jnp.dot(bdist_ref[...].reshape(1, D3), W1, preferred_element_type=f32)
                + jnp.dot(emb_ref[N:N + 1, :], W2, preferred_element_type=f32)
                + jnp.dot(bdens_ref[...].reshape(1, D3), W3, preferred_element_type=f32)
                + bout_ref[...].reshape(1, ED))
        cd3 = cd_ref[...][:, :, None]
        ld3 = ld_ref[...][:, :, None]
        out_ref[...] = (cd3 * v1.reshape(1, 1, ED)
                        + ld3 * v2.reshape(1, 1, ED)
                        + cvec.reshape(1, 1, ED))

    return pl.pallas_call(
        body,
        grid=grid,
        in_specs=[
            pl.BlockSpec((BB, N), lambda i: (i, 0)),
            pl.BlockSpec((BB, N), lambda i: (i, 0)),
            pl.BlockSpec((1, D3), lambda i: (0, 0)),
            pl.BlockSpec((D3,), lambda i: (0,)),
            pl.BlockSpec((50, D3), lambda i: (0, 0)),
            pl.BlockSpec((1, D3), lambda i: (0, 0)),
            pl.BlockSpec((D3,), lambda i: (0,)),
            pl.BlockSpec((ED, ED), lambda i: (0, 0)),
            pl.BlockSpec((ED,), lambda i: (0,)),
        ],
        out_specs=pl.BlockSpec((BB, N, ED), lambda i: (i, 0, 0)),
        out_shape=jax.ShapeDtypeStruct((B, N, ED), jnp.float32),
        compiler_params=pltpu.CompilerParams(
            dimension_semantics=("arbitrary",)),
    )


def kernel(points, W_dist, b_dist, emb_count, W_dens, b_dens, W_out, b_out):
    B = points.shape[0]
    pts_t = jnp.transpose(points, (0, 2, 1)).reshape(B * 3 * N)
    cd, ld = _make_sc_scalars(B)(pts_t)
    return _make_tc_expand(B)(cd, ld, W_dist, b_dist, emb_count, W_dens,
                              b_dens, W_out, b_out)


# speedup vs baseline: 1.7611x; 1.1455x over previous
import functools

import jax
import jax.numpy as jnp
import numpy as np
from jax import lax
from jax.experimental import pallas as pl
from jax.experimental.pallas import tpu as pltpu
from jax.experimental.pallas import tpu_sc as plsc

N = 48
D3 = 128
ED = 384
NW = 32
CH = 16
_BIG = np.float32(3.0e38)


def _sqrt16(x):
    xc = jnp.maximum(x, np.float32(1e-30))
    i = plsc.bitcast(xc, jnp.int32)
    i = np.int32(0x5F3759DF) - lax.shift_right_logical(i, 1)
    y = plsc.bitcast(i, jnp.float32)
    half = np.float32(0.5) * xc
    for _ in range(3):
        y = y * (np.float32(1.5) - half * y * y)
    return x * y


def _bcast_lane(v, idxv):
    dn = lax.GatherDimensionNumbers(
        offset_dims=(), collapsed_slice_dims=(0,), start_index_map=(0,))
    return lax.gather(v, idxv[:, None], dn, (1,),
                      mode=lax.GatherScatterMode.PROMISE_IN_BOUNDS)


@functools.lru_cache(maxsize=None)
def _make_sc_scalars(B, npart=1, part=0):
    Bp = B // npart
    CPW = Bp // NW
    NCHUNK = CPW // CH
    gbase = part * Bp
    mesh = plsc.VectorSubcoreMesh(core_axis_name="c", subcore_axis_name="s")

    @functools.partial(
        pl.kernel,
        out_type=(jax.ShapeDtypeStruct((Bp, N), jnp.float32),
                  jax.ShapeDtypeStruct((Bp, N), jnp.float32)),
        mesh=mesh,
        scratch_types=[
            pltpu.VMEM((CH * N * 3,), jnp.float32),
            pltpu.VMEM((CH, N), jnp.float32),
            pltpu.VMEM((CH, N), jnp.float32),
        ],
        compiler_params=pltpu.CompilerParams(needs_layout_passes=False),
    )
    def sc_scalars(pts_hbm, cd_hbm, ld_hbm, pts_v, cd_v, ld_v):
        wid = lax.axis_index("s") * 2 + lax.axis_index("c")
        base = wid * CPW
        iota = lax.iota(jnp.int32, 16)
        zero16 = iota * 0

        def cloud_body(ci, _):
            cbase = ci * (N * 3)
            xs, ys, zs = [], [], []
            for c in range(3):
                off = cbase + 16 * c
                xs.append(pts_v[pl.ds(off, 16)])
                ys.append(pts_v[pl.ds(off + N, 16)])
                zs.append(pts_v[pl.ds(off + 2 * N, 16)])
            inv_n = np.float32(1.0 / N)
            cx = jnp.sum(xs[0] + xs[1] + xs[2]) * inv_n
            cy = jnp.sum(ys[0] + ys[1] + ys[2]) * inv_n
            cz = jnp.sum(zs[0] + zs[1] + zs[2]) * inv_n
            big = zero16.astype(jnp.float32) + _BIG
            m1 = [big, big, big]
            m2 = [big, big, big]
            m3 = [big, big, big]
            for jc in range(3):
                for jl in range(16):
                    jidx = zero16 + jl
                    xj = _bcast_lane(xs[jc], jidx)
                    yj = _bcast_lane(ys[jc], jidx)
                    zj = _bcast_lane(zs[jc], jidx)
                    for ic in range(3):
                        dx = xs[ic] - xj
                        dy = ys[ic] - yj
                        dz = zs[ic] - zj
                        d = dx * dx + dy * dy + dz * dz
                        if ic == jc:
                            d = jnp.where(iota == jl, _BIG, d)
                        lo = jnp.minimum(m1[ic], d)
                        hi = jnp.maximum(m1[ic], d)
                        m1[ic] = lo
                        lo2 = jnp.minimum(m2[ic], hi)
                        hi2 = jnp.maximum(m2[ic], hi)
                        m2[ic] = lo2
                        m3[ic] = jnp.minimum(m3[ic], hi2)
            third = np.float32(1.0 / 3.0)
            for ic in range(3):
                dxc = xs[ic] - cx
                dyc = ys[ic] - cy
                dzc = zs[ic] - cz
                cdv = _sqrt16(dxc * dxc + dyc * dyc + dzc * dzc)
                ldv = (_sqrt16(m1[ic]) + _sqrt16(m2[ic]) + _sqrt16(m3[ic])) * third
                cd_v[ci, pl.ds(16 * ic, 16)] = cdv
                ld_v[ci, pl.ds(16 * ic, 16)] = ldv
            return 0

        def chunk_body(ck, _):
            b0 = base + ck * CH
            pltpu.sync_copy(
                pts_hbm.at[pl.ds((gbase + b0) * (N * 3), CH * N * 3)], pts_v)
            lax.fori_loop(0, CH, cloud_body, 0, unroll=False)
            pltpu.sync_copy(cd_v, cd_hbm.at[pl.ds(b0, CH)])
            pltpu.sync_copy(ld_v, ld_hbm.at[pl.ds(b0, CH)])
            return 0

        lax.fori_loop(0, NCHUNK, chunk_body, 0, unroll=False)

    return sc_scalars


@functools.lru_cache(maxsize=None)
def _make_tc_expand(B, npart=1, part=0, aliased=False, BB=32):
    Bp = B // npart
    steps = Bp // BB
    grid = (steps,)

    def body(cd_ref, ld_ref, wdist_ref, bdist_ref, emb_ref, wdens_ref,
             bdens_ref, wout_ref, bout_ref, *refs):
        out_ref = refs[-1]
        W = wout_ref[...]
        W1 = W[0:D3, :]
        W2 = W[D3:2 * D3, :]
        W3 = W[2 * D3:, :]
        f32 = jnp.float32
        v1 = jnp.dot(wdist_ref[...], W1, preferred_element_type=f32)
        v2 = jnp.dot(wdens_ref[...], W3, preferred_element_type=f32)
        cvec = (jnp.dot(bdist_ref[...].reshape(1, D3), W1, preferred_element_type=f32)
                + jnp.dot(emb_ref[N:N + 1, :], W2, preferred_element_type=f32)
                + jnp.dot(bdens_ref[...].reshape(1, D3), W3, preferred_element_type=f32)
                + bout_ref[...].reshape(1, ED))
        cd3 = cd_ref[...][:, :, None]
        ld3 = ld_ref[...][:, :, None]
        out_ref[...] = (cd3 * v1.reshape(1, 1, ED)
                        + ld3 * v2.reshape(1, 1, ED)
                        + cvec.reshape(1, 1, ED))

    in_specs = [
        pl.BlockSpec((BB, N), lambda i: (i, 0)),
        pl.BlockSpec((BB, N), lambda i: (i, 0)),
        pl.BlockSpec((1, D3), lambda i: (0, 0)),
        pl.BlockSpec((D3,), lambda i: (0,)),
        pl.BlockSpec((50, D3), lambda i: (0, 0)),
        pl.BlockSpec((1, D3), lambda i: (0, 0)),
        pl.BlockSpec((D3,), lambda i: (0,)),
        pl.BlockSpec((ED, ED), lambda i: (0, 0)),
        pl.BlockSpec((ED,), lambda i: (0,)),
    ]
    kwargs = {}
    if aliased:
        in_specs.append(pl.BlockSpec(memory_space=pltpu.MemorySpace.HBM))
        kwargs["input_output_aliases"] = {9: 0}
    off = part * steps
    return pl.pallas_call(
        body,
        grid=grid,
        in_specs=in_specs,
        out_specs=pl.BlockSpec((BB, N, ED), lambda i: (i + off, 0, 0)),
        out_shape=jax.ShapeDtypeStruct((B, N, ED), jnp.float32),
        compiler_params=pltpu.CompilerParams(
            dimension_semantics=("arbitrary",)),
        **kwargs,
    )


def kernel(points, W_dist, b_dist, emb_count, W_dens, b_dens, W_out, b_out):
    B = points.shape[0]
    pts_t = jnp.transpose(points, (0, 2, 1)).reshape(B * 3 * N)
    w = (W_dist, b_dist, emb_count, W_dens, b_dens, W_out, b_out)
    cd0, ld0 = _make_sc_scalars(B, 2, 0)(pts_t)
    cd1, ld1 = _make_sc_scalars(B, 2, 1)(pts_t)
    out0 = _make_tc_expand(B, 2, 0)(cd0, ld0, *w)
    return _make_tc_expand(B, 2, 1, aliased=True)(cd1, ld1, *w, out0)


# speedup vs baseline: 1.9993x; 1.1353x over previous
import functools

import jax
import jax.numpy as jnp
import numpy as np
from jax import lax
from jax.experimental import pallas as pl
from jax.experimental.pallas import tpu as pltpu
from jax.experimental.pallas import tpu_sc as plsc

N = 48
D3 = 128
ED = 384
NW = 32
CH = 16
_BIG = np.float32(3.0e38)


def _sqrt16(x):
    xc = jnp.maximum(x, np.float32(1e-30))
    i = plsc.bitcast(xc, jnp.int32)
    i = np.int32(0x5F3759DF) - lax.shift_right_logical(i, 1)
    y = plsc.bitcast(i, jnp.float32)
    half = np.float32(0.5) * xc
    for _ in range(3):
        y = y * (np.float32(1.5) - half * y * y)
    return x * y


def _bcast_lane(v, idxv):
    dn = lax.GatherDimensionNumbers(
        offset_dims=(), collapsed_slice_dims=(0,), start_index_map=(0,))
    return lax.gather(v, idxv[:, None], dn, (1,),
                      mode=lax.GatherScatterMode.PROMISE_IN_BOUNDS)


@functools.lru_cache(maxsize=None)
def _make_sc_scalars(B, npart=1, part=0):
    Bp = B // npart
    CPW = Bp // NW
    NCHUNK = CPW // CH
    gbase = part * Bp
    mesh = plsc.VectorSubcoreMesh(core_axis_name="c", subcore_axis_name="s")

    @functools.partial(
        pl.kernel,
        out_type=(jax.ShapeDtypeStruct((Bp, N), jnp.float32),
                  jax.ShapeDtypeStruct((Bp, N), jnp.float32)),
        mesh=mesh,
        scratch_types=[
            pltpu.VMEM((CH * N * 3,), jnp.float32),
            pltpu.VMEM((CH, N), jnp.float32),
            pltpu.VMEM((CH, N), jnp.float32),
        ],
        compiler_params=pltpu.CompilerParams(needs_layout_passes=False),
    )
    def sc_scalars(pts_hbm, cd_hbm, ld_hbm, pts_v, cd_v, ld_v):
        wid = lax.axis_index("s") * 2 + lax.axis_index("c")
        base = wid * CPW
        iota = lax.iota(jnp.int32, 16)
        zero16 = iota * 0

        def cloud_body(ci, _):
            cbase = ci * (N * 3)
            xs, ys, zs = [], [], []
            for c in range(3):
                off = cbase + 16 * c
                xs.append(pts_v[pl.ds(off, 16)])
                ys.append(pts_v[pl.ds(off + N, 16)])
                zs.append(pts_v[pl.ds(off + 2 * N, 16)])
            inv_n = np.float32(1.0 / N)
            cx = jnp.sum(xs[0] + xs[1] + xs[2]) * inv_n
            cy = jnp.sum(ys[0] + ys[1] + ys[2]) * inv_n
            cz = jnp.sum(zs[0] + zs[1] + zs[2]) * inv_n
            big = zero16.astype(jnp.float32) + _BIG
            m1 = [big, big, big]
            m2 = [big, big, big]
            m3 = [big, big, big]
            for jc in range(3):
                for jl in range(16):
                    jidx = zero16 + jl
                    xj = _bcast_lane(xs[jc], jidx)
                    yj = _bcast_lane(ys[jc], jidx)
                    zj = _bcast_lane(zs[jc], jidx)
                    for ic in range(3):
                        dx = xs[ic] - xj
                        dy = ys[ic] - yj
                        dz = zs[ic] - zj
                        d = dx * dx + dy * dy + dz * dz
                        if ic == jc:
                            d = jnp.where(iota == jl, _BIG, d)
                        lo = jnp.minimum(m1[ic], d)
                        hi = jnp.maximum(m1[ic], d)
                        m1[ic] = lo
                        lo2 = jnp.minimum(m2[ic], hi)
                        hi2 = jnp.maximum(m2[ic], hi)
                        m2[ic] = lo2
                        m3[ic] = jnp.minimum(m3[ic], hi2)
            third = np.float32(1.0 / 3.0)
            for ic in range(3):
                dxc = xs[ic] - cx
                dyc = ys[ic] - cy
                dzc = zs[ic] - cz
                cdv = _sqrt16(dxc * dxc + dyc * dyc + dzc * dzc)
                ldv = (_sqrt16(m1[ic]) + _sqrt16(m2[ic]) + _sqrt16(m3[ic])) * third
                cd_v[ci, pl.ds(16 * ic, 16)] = cdv
                ld_v[ci, pl.ds(16 * ic, 16)] = ldv
            return 0

        def chunk_body(ck, _):
            b0 = base + ck * CH
            pltpu.sync_copy(
                pts_hbm.at[pl.ds((gbase + b0) * (N * 3), CH * N * 3)], pts_v)
            lax.fori_loop(0, CH, cloud_body, 0, unroll=False)
            pltpu.sync_copy(cd_v, cd_hbm.at[pl.ds(b0, CH)])
            pltpu.sync_copy(ld_v, ld_hbm.at[pl.ds(b0, CH)])
            return 0

        lax.fori_loop(0, NCHUNK, chunk_body, 0, unroll=False)

    return sc_scalars


@functools.lru_cache(maxsize=None)
def _make_tc_expand(B, npart=1, part=0, aliased=False, BB=32):
    Bp = B // npart
    steps = Bp // BB
    grid = (steps,)

    def body(cd_ref, ld_ref, wdist_ref, bdist_ref, emb_ref, wdens_ref,
             bdens_ref, wout_ref, bout_ref, *refs):
        out_ref = refs[-1]
        W = wout_ref[...]
        W1 = W[0:D3, :]
        W2 = W[D3:2 * D3, :]
        W3 = W[2 * D3:, :]
        f32 = jnp.float32
        v1 = jnp.dot(wdist_ref[...], W1, preferred_element_type=f32)
        v2 = jnp.dot(wdens_ref[...], W3, preferred_element_type=f32)
        cvec = (jnp.dot(bdist_ref[...].reshape(1, D3), W1, preferred_element_type=f32)
                + jnp.dot(emb_ref[N:N + 1, :], W2, preferred_element_type=f32)
                + jnp.dot(bdens_ref[...].reshape(1, D3), W3, preferred_element_type=f32)
                + bout_ref[...].reshape(1, ED))
        cd3 = cd_ref[...][:, :, None]
        ld3 = ld_ref[...][:, :, None]
        out_ref[...] = (cd3 * v1.reshape(1, 1, ED)
                        + ld3 * v2.reshape(1, 1, ED)
                        + cvec.reshape(1, 1, ED))

    in_specs = [
        pl.BlockSpec((BB, N), lambda i: (i, 0)),
        pl.BlockSpec((BB, N), lambda i: (i, 0)),
        pl.BlockSpec((1, D3), lambda i: (0, 0)),
        pl.BlockSpec((D3,), lambda i: (0,)),
        pl.BlockSpec((50, D3), lambda i: (0, 0)),
        pl.BlockSpec((1, D3), lambda i: (0, 0)),
        pl.BlockSpec((D3,), lambda i: (0,)),
        pl.BlockSpec((ED, ED), lambda i: (0, 0)),
        pl.BlockSpec((ED,), lambda i: (0,)),
    ]
    kwargs = {}
    if aliased:
        in_specs.append(pl.BlockSpec(memory_space=pltpu.MemorySpace.HBM))
        kwargs["input_output_aliases"] = {9: 0}
    off = part * steps
    return pl.pallas_call(
        body,
        grid=grid,
        in_specs=in_specs,
        out_specs=pl.BlockSpec((BB, N, ED), lambda i: (i + off, 0, 0)),
        out_shape=jax.ShapeDtypeStruct((B, N, ED), jnp.float32),
        compiler_params=pltpu.CompilerParams(
            dimension_semantics=("arbitrary",)),
        **kwargs,
    )


def kernel(points, W_dist, b_dist, emb_count, W_dens, b_dens, W_out, b_out):
    B = points.shape[0]
    w = (W_dist, b_dist, emb_count, W_dens, b_dens, W_out, b_out)
    NP = 4
    Bp = B // NP
    scalars = []
    for k in range(NP):
        pts_k = jnp.transpose(points[k * Bp:(k + 1) * Bp],
                              (0, 2, 1)).reshape(Bp * 3 * N)
        scalars.append(_make_sc_scalars(Bp)(pts_k))
    out = None
    for k in range(NP):
        cdk, ldk = scalars[k]
        if k == 0:
            out = _make_tc_expand(B, NP, 0)(cdk, ldk, *w)
        else:
            out = _make_tc_expand(B, NP, k, aliased=True)(cdk, ldk, *w, out)
    return out
